# baseline (speedup 1.0000x reference)
import functools

import jax
import jax.numpy as jnp
from jax import lax
from jax.experimental import pallas as pl
from jax.experimental.pallas import tpu as pltpu
from jax.experimental.pallas import tpu_sc as plsc

AUDIO_DIM = 128
TEXT_DIM = 768
EPS = 1e-06

BATCH = 4096
HIST = 20

NC = 2
NS = 16
NW = NC * NS
CHUNK = 128
GROUP = 4
N_TOKENS = BATCH * HIST

N_CH = 5
C_TOKENS = N_TOKENS // N_CH
B_PER_W = C_TOKENS // NW

ROWS = 4096
C_STEPS = C_TOKENS // ROWS


def _sc_gather(table, idx3):
    mesh = plsc.VectorSubcoreMesh(core_axis_name="c", subcore_axis_name="s")

    @functools.partial(
        pl.kernel,
        out_type=jax.ShapeDtypeStruct((C_TOKENS, AUDIO_DIM), jnp.float32),
        mesh=mesh,
        scratch_types=[
            pltpu.VMEM((GROUP, CHUNK), jnp.int32),
            pltpu.VMEM((GROUP * CHUNK, AUDIO_DIM), jnp.float32),
            pltpu.SemaphoreType.DMA,
        ],
    )
    def k(table_hbm, idx_hbm, out_hbm, idx_v, rows_v, sem):
        wid = lax.axis_index("s") * NC + lax.axis_index("c")
        pltpu.sync_copy(idx_hbm.at[wid], idx_v)
        copies = [
            pltpu.async_copy(
                table_hbm.at[idx_v.at[b]],
                rows_v.at[pl.ds(b * CHUNK, CHUNK)],
                sem,
            )
            for b in range(GROUP)
        ]
        for cp in copies:
            cp.wait()
        pltpu.sync_copy(rows_v, out_hbm.at[pl.ds(wid * B_PER_W, B_PER_W)])

    return k(table, idx3)


def _dense_body(x_ref, s_ref, w_ref, o_ref):
    xv = x_ref[...]
    var = jnp.mean(xv * xv, axis=-1, keepdims=True)
    xn = xv * lax.rsqrt(var + EPS) * s_ref[...]
    p = jnp.dot(xn, w_ref[...], preferred_element_type=jnp.float32)
    var2 = jnp.mean(p * p, axis=-1, keepdims=True)
    o_ref[...] = p * lax.rsqrt(var2 + EPS)


def _tc_dense_first(x, scale, w):
    return pl.pallas_call(
        _dense_body,
        grid=(C_STEPS,),
        in_specs=[
            pl.BlockSpec((ROWS, AUDIO_DIM), lambda i: (i, 0)),
            pl.BlockSpec((1, AUDIO_DIM), lambda i: (0, 0)),
            pl.BlockSpec((AUDIO_DIM, TEXT_DIM), lambda i: (0, 0)),
        ],
        out_specs=pl.BlockSpec((ROWS, TEXT_DIM), lambda i: (i, 0)),
        out_shape=jax.ShapeDtypeStruct((N_TOKENS, TEXT_DIM), jnp.float32),
    )(x, scale, w)


def _tc_dense_chunk(out, x, scale, w, c):
    def body(o_in_ref, x_ref, s_ref, w_ref, o_ref):
        del o_in_ref
        _dense_body(x_ref, s_ref, w_ref, o_ref)

    return pl.pallas_call(
        body,
        grid=(C_STEPS,),
        in_specs=[
            pl.BlockSpec(memory_space=pl.ANY),
            pl.BlockSpec((ROWS, AUDIO_DIM), lambda i: (i, 0)),
            pl.BlockSpec((1, AUDIO_DIM), lambda i: (0, 0)),
            pl.BlockSpec((AUDIO_DIM, TEXT_DIM), lambda i: (0, 0)),
        ],
        out_specs=pl.BlockSpec((ROWS, TEXT_DIM), lambda i: (c * C_STEPS + i, 0)),
        out_shape=jax.ShapeDtypeStruct((N_TOKENS, TEXT_DIM), jnp.float32),
        input_output_aliases={0: 0},
    )(out, x, scale, w)


def kernel(input_ids, table, norm_scale, proj_w):
    batch, hist = input_ids.shape
    flat = input_ids.T.astype(jnp.int32).reshape(-1)
    scale = norm_scale.reshape(1, AUDIO_DIM)
    w = proj_w.T

    gathered = [
        _sc_gather(
            table,
            lax.dynamic_slice_in_dim(flat, c * C_TOKENS, C_TOKENS)
            .reshape(NW, GROUP, CHUNK),
        )
        for c in range(N_CH)
    ]
    out = _tc_dense_first(gathered[0], scale, w)
    for c in range(1, N_CH):
        out = _tc_dense_chunk(out, gathered[c], scale, w, c)
    return out.reshape(hist, batch, TEXT_DIM).transpose(1, 0, 2)


# speedup vs baseline: 1.0257x; 1.0257x over previous
import functools

import jax
import jax.numpy as jnp
from jax import lax
from jax.experimental import pallas as pl
from jax.experimental.pallas import tpu as pltpu
from jax.experimental.pallas import tpu_sc as plsc

AUDIO_DIM = 128
TEXT_DIM = 768
EPS = 1e-06

BATCH = 4096
HIST = 20

NC = 2
NS = 16
NW = NC * NS
CHUNK = 128
GROUP = 4
N_TOKENS = BATCH * HIST
B_PER_W = N_TOKENS // NW
N_CHUNKS = B_PER_W // CHUNK
N_GROUPS = N_CHUNKS // GROUP

ROWS = 4096


def _sc_gather(table, idx3):
    mesh = plsc.VectorSubcoreMesh(core_axis_name="c", subcore_axis_name="s")

    @functools.partial(
        pl.kernel,
        out_type=jax.ShapeDtypeStruct((N_TOKENS, AUDIO_DIM), jnp.float32),
        mesh=mesh,
        scratch_types=[
            pltpu.VMEM((N_CHUNKS, CHUNK), jnp.int32),
            pltpu.VMEM((GROUP * CHUNK, AUDIO_DIM), jnp.float32),
            pltpu.SemaphoreType.DMA,
        ],
    )
    def k(table_hbm, idx_hbm, out_hbm, idx_v, rows_v, sem):
        wid = lax.axis_index("s") * NC + lax.axis_index("c")
        base = wid * B_PER_W
        pltpu.sync_copy(idx_hbm.at[wid], idx_v)
        for g in range(N_GROUPS):
            copies = [
                pltpu.async_copy(
                    table_hbm.at[idx_v.at[g * GROUP + b]],
                    rows_v.at[pl.ds(b * CHUNK, CHUNK)],
                    sem,
                )
                for b in range(GROUP)
            ]
            for cp in copies:
                cp.wait()
            pltpu.sync_copy(
                rows_v, out_hbm.at[pl.ds(base + g * GROUP * CHUNK, GROUP * CHUNK)]
            )

    return k(table, idx3)


def _tc_dense(x, scale, w):
    grid = (N_TOKENS // ROWS,)

    def body(x_ref, s_ref, w_ref, o_ref):
        xv = x_ref[...]
        var = jnp.mean(xv * xv, axis=-1, keepdims=True)
        xn = xv * lax.rsqrt(var + EPS) * s_ref[...]
        p = jnp.dot(xn, w_ref[...], preferred_element_type=jnp.float32)
        var2 = jnp.mean(p * p, axis=-1, keepdims=True)
        o_ref[...] = p * lax.rsqrt(var2 + EPS)

    return pl.pallas_call(
        body,
        grid=grid,
        in_specs=[
            pl.BlockSpec((ROWS, AUDIO_DIM), lambda i: (i, 0)),
            pl.BlockSpec((1, AUDIO_DIM), lambda i: (0, 0)),
            pl.BlockSpec((AUDIO_DIM, TEXT_DIM), lambda i: (0, 0)),
        ],
        out_specs=pl.BlockSpec((ROWS, TEXT_DIM), lambda i: (i, 0)),
        out_shape=jax.ShapeDtypeStruct((N_TOKENS, TEXT_DIM), jnp.float32),
    )(x, scale, w)


def kernel(input_ids, table, norm_scale, proj_w):
    batch, hist = input_ids.shape
    ids_t = input_ids.T.astype(jnp.int32)
    idx3 = ids_t.reshape(NW, N_CHUNKS, CHUNK)
    gathered = _sc_gather(table, idx3)
    out = _tc_dense(gathered, norm_scale.reshape(1, AUDIO_DIM), proj_w.T)
    return out.reshape(hist, batch, TEXT_DIM).transpose(1, 0, 2)
